# baseline (speedup 1.0000x reference)
import jax
import jax.numpy as jnp
from jax.experimental import pallas as pl
from jax.experimental.pallas import tpu as pltpu

DIM = 512
HID = 512
E = 16
K = 2
LIMIT = 7.0
TOK = 64
NC = 4
C1 = HID // NC
C2 = DIM // NC


def _copies(w1_hbm, w2_hbm, w1_buf, w2_buf, sem, e, slot):
    cps = []
    for c in range(NC):
        cps.append(pltpu.make_async_copy(
            w1_hbm.at[e, pl.ds(c * C1, C1)],
            w1_buf.at[slot, pl.ds(c * C1, C1)],
            sem.at[slot, c]))
        cps.append(pltpu.make_async_copy(
            w2_hbm.at[e, pl.ds(c * C2, C2)],
            w2_buf.at[slot, pl.ds(c * C2, C2)],
            sem.at[slot, NC + c]))
    return cps


def _moe_step(x_ref, gw_ref, gb_ref, w1_hbm, w2_hbm, bias_ref, out_ref,
              w1_buf, w2_buf, sem):
    i = pl.program_id(0)
    slot = jax.lax.rem(i, 2)

    @pl.when(i == 0)
    def _first():
        for cp in _copies(w1_hbm, w2_hbm, w1_buf, w2_buf, sem, 0, 0):
            cp.start()

    @pl.when(i + 1 < E)
    def _prefetch():
        nxt = jax.lax.rem(i + 1, 2)
        for cp in _copies(w1_hbm, w2_hbm, w1_buf, w2_buf, sem, i + 1, nxt):
            cp.start()

    x = x_ref[...]

    g = jnp.dot(x, gw_ref[...], preferred_element_type=jnp.float32)
    g = g + gb_ref[...]
    iota = jax.lax.broadcasted_iota(jnp.int32, (TOK, E), 1)
    m1 = jnp.max(g, axis=1, keepdims=True)
    idx1 = jnp.min(jnp.where(g == m1, iota, E), axis=1, keepdims=True)
    g2 = jnp.where(iota == idx1, -jnp.inf, g)
    m2 = jnp.max(g2, axis=1, keepdims=True)
    idx2 = jnp.min(jnp.where((g2 == m2) & (iota != idx1), iota, E),
                   axis=1, keepdims=True)
    z = jnp.exp(m2 - m1)
    p1 = 1.0 / (1.0 + z)
    p2 = 1.0 - p1
    p = jnp.where(idx1 == i, p1, jnp.where(idx2 == i, p2, 0.0))

    for cp in _copies(w1_hbm, w2_hbm, w1_buf, w2_buf, sem, i, slot):
        cp.wait()

    cdims = (((1,), (1,)), ((), ()))
    wp = w1_buf[slot]
    bias = bias_ref[0]
    hg = jax.lax.dot_general(x, wp[:, :DIM], cdims,
                             preferred_element_type=jnp.float32)
    hl = jax.lax.dot_general(x, wp[:, DIM:], cdims,
                             preferred_element_type=jnp.float32)
    hg = jnp.minimum(hg + bias[:, :HID], LIMIT)
    hl = jnp.clip(hl + bias[:, HID:2 * HID], -LIMIT, LIMIT)
    act = hg * jax.nn.sigmoid(1.702 * hg) * (hl + 1.0)
    y = jax.lax.dot_general(act, w2_buf[slot], cdims,
                            preferred_element_type=jnp.float32)
    y = y + bias[:, 2 * HID:]

    contrib = p * y

    @pl.when(i == 0)
    def _init():
        out_ref[...] = contrib

    @pl.when(i != 0)
    def _acc():
        out_ref[...] = out_ref[...] + contrib


def kernel(x, gate_w, gate_b, w1, b1, w2, b2):
    w1r = w1.reshape(E, HID, 2 * DIM)
    bias = jnp.concatenate([b1[:, 0::2], b1[:, 1::2], b2], axis=1)
    bias = bias.reshape(E, 1, 2 * HID + DIM)
    gb = gate_b.reshape(1, E)

    full = lambda i: (0, 0)
    per_i3 = lambda i: (i, 0, 0)
    out = pl.pallas_call(
        _moe_step,
        grid=(E,),
        in_specs=[
            pl.BlockSpec((TOK, DIM), full),
            pl.BlockSpec((DIM, E), full),
            pl.BlockSpec((1, E), full),
            pl.BlockSpec(memory_space=pltpu.HBM),
            pl.BlockSpec(memory_space=pltpu.HBM),
            pl.BlockSpec((1, 1, 2 * HID + DIM), per_i3),
        ],
        out_specs=pl.BlockSpec((TOK, DIM), full),
        out_shape=jax.ShapeDtypeStruct((TOK, DIM), jnp.float32),
        scratch_shapes=[
            pltpu.VMEM((2, HID, 2 * DIM), jnp.float32),
            pltpu.VMEM((2, DIM, HID), jnp.float32),
            pltpu.SemaphoreType.DMA((2, 2 * NC)),
        ],
    )(x, gate_w, gb, w1r, w2, bias)
    return out


# speedup vs baseline: 1.0783x; 1.0783x over previous
import jax
import jax.numpy as jnp
from jax.experimental import pallas as pl

DIM = 512
HID = 512
E = 16
K = 2
LIMIT = 7.0
TOK = 64
EPB = 2


def _moe_step(x_ref, gw_ref, gb_ref, w1_ref, w2_ref, bias_ref, out_ref):
    i = pl.program_id(0)
    x = x_ref[...]

    g = jnp.dot(x, gw_ref[...], preferred_element_type=jnp.float32)
    g = g + gb_ref[...]
    iota = jax.lax.broadcasted_iota(jnp.int32, (TOK, E), 1)
    m1 = jnp.max(g, axis=1, keepdims=True)
    idx1 = jnp.min(jnp.where(g == m1, iota, E), axis=1, keepdims=True)
    g2 = jnp.where(iota == idx1, -jnp.inf, g)
    m2 = jnp.max(g2, axis=1, keepdims=True)
    idx2 = jnp.min(jnp.where((g2 == m2) & (iota != idx1), iota, E),
                   axis=1, keepdims=True)
    z = jnp.exp(m2 - m1)
    p1 = 1.0 / (1.0 + z)
    p2 = 1.0 - p1

    cdims = (((1,), (1,)), ((), ()))
    acc = jnp.zeros((TOK, DIM), jnp.float32)
    for j in range(EPB):
        e = i * EPB + j
        p = jnp.where(idx1 == e, p1, jnp.where(idx2 == e, p2, 0.0))
        wp = w1_ref[j]
        bias = bias_ref[j]
        hg = jax.lax.dot_general(x, wp[:, :DIM], cdims,
                                 preferred_element_type=jnp.float32)
        hl = jax.lax.dot_general(x, wp[:, DIM:], cdims,
                                 preferred_element_type=jnp.float32)
        hg = jnp.minimum(hg + bias[:, :HID], LIMIT)
        hl = jnp.clip(hl + bias[:, HID:2 * HID], -LIMIT, LIMIT)
        act = hg * jax.nn.sigmoid(1.702 * hg) * (hl + 1.0)
        y = jax.lax.dot_general(act, w2_ref[j], cdims,
                                preferred_element_type=jnp.float32)
        y = y + bias[:, 2 * HID:]
        acc = acc + p * y

    @pl.when(i == 0)
    def _init():
        out_ref[...] = acc

    @pl.when(i != 0)
    def _acc():
        out_ref[...] = out_ref[...] + acc


def kernel(x, gate_w, gate_b, w1, b1, w2, b2):
    w1r = w1.reshape(E, HID, 2 * DIM)
    bias = jnp.concatenate([b1[:, 0::2], b1[:, 1::2], b2], axis=1)
    bias = bias.reshape(E, 1, 2 * HID + DIM)
    gb = gate_b.reshape(1, E)

    full = lambda i: (0, 0)
    per_i3 = lambda i: (i, 0, 0)
    out = pl.pallas_call(
        _moe_step,
        grid=(E // EPB,),
        in_specs=[
            pl.BlockSpec((TOK, DIM), full),
            pl.BlockSpec((DIM, E), full),
            pl.BlockSpec((1, E), full),
            pl.BlockSpec((EPB, HID, 2 * DIM), per_i3),
            pl.BlockSpec((EPB, DIM, HID), per_i3),
            pl.BlockSpec((EPB, 1, 2 * HID + DIM), per_i3),
        ],
        out_specs=pl.BlockSpec((TOK, DIM), full),
        out_shape=jax.ShapeDtypeStruct((TOK, DIM), jnp.float32),
    )(x, gate_w, gb, w1r, w2, bias)
    return out
